# baseline (speedup 1.0000x reference)
import functools

import jax
import jax.numpy as jnp
from jax import lax
from jax.experimental import pallas as pl
from jax.experimental.pallas import tpu as pltpu
from jax.experimental.pallas import tpu_sc as plsc

NC = 2
NS = 16
NW = NC * NS
L = 16

B = 8
M = 1024 * 1024
T = 512
F = 2

K = 8192
N = 1024
PER_W = M // NW
CH = 8192
CROWS = CH // N
NCHUNK = PER_W // CH
U = 4
BINS_PAD = T + L
HIST_LEN = 1056
NB_LUT = 8


def _sc_histogram(inp_flat, bins, w_flat):
    mesh = plsc.VectorSubcoreMesh(
        core_axis_name="c", subcore_axis_name="s", num_cores=NC,
        num_subcores=NS)

    @functools.partial(
        pl.kernel,
        out_type=[
            jax.ShapeDtypeStruct((NW, B, HIST_LEN), jnp.float32),
            jax.ShapeDtypeStruct((NW, B, 2 * L), jnp.float32),
        ],
        mesh=mesh,
        compiler_params=pltpu.CompilerParams(needs_layout_passes=False),
        scratch_types=[
            pltpu.VMEM((CROWS, N), jnp.float32),
            pltpu.VMEM((CROWS, N), jnp.float32),
            pltpu.VMEM((CROWS, N * F), jnp.float32),
            pltpu.VMEM((CROWS, N * F), jnp.float32),
            pltpu.VMEM((BINS_PAD,), jnp.float32),
            pltpu.VMEM((K,), jnp.float32),
            pltpu.VMEM((K,), jnp.int32),
            pltpu.VMEM((HIST_LEN,), jnp.float32),
            pltpu.VMEM((2 * L,), jnp.float32),
            pltpu.VMEM_SHARED((B, K), jnp.int32),
            pltpu.SemaphoreType.DMA,
            pltpu.SemaphoreType.DMA,
            pltpu.SemaphoreType.DMA,
            pltpu.SemaphoreType.DMA,
        ],
    )
    def k(inp4_hbm, bins_hbm, w4_hbm, out_hist, out_min,
          vbuf0, vbuf1, wbuf0, wbuf1, binspad, bcnt, bcum, hist, minbuf,
          lut_shared, sv0, sv1, sw0, sw1):
        cid = lax.axis_index("c")
        sid = lax.axis_index("s")
        wid = sid * NC + cid
        wbase = wid * PER_W
        lanes = lax.iota(jnp.int32, L)
        zf16 = jnp.zeros((L,), jnp.float32)
        ones_f = jnp.ones((L,), jnp.float32)
        one_i = jnp.ones((L,), jnp.int32)
        zero_i = jnp.zeros((L,), jnp.int32)
        vbufs = (vbuf0, vbuf1)
        wbufs = (wbuf0, wbuf1)
        svs = (sv0, sv1)
        sws = (sw0, sw1)

        @pl.when(sid < NB_LUT)
        def _build():
            def zero_lut(i, _):
                bcnt[pl.ds(i * L, L)] = zf16
                return 0
            lax.fori_loop(0, K // L, zero_lut, 0)

            pltpu.sync_copy(bins_hbm.at[sid], binspad.at[pl.ds(0, T)])
            binspad[pl.ds(T, L)] = jnp.full((L,), 2.0, jnp.float32)

            def bin_cells(i, _):
                bv = binspad[pl.ds(i * L, L)]
                cb = jnp.minimum((bv * K).astype(jnp.int32), K - 1)
                plsc.addupdate_scatter(bcnt, [cb], ones_f)
                return 0
            lax.fori_loop(0, T // L, bin_cells, 0)

            def prefix(i, tot):
                vec = bcnt[pl.ds(i * L, L)]
                cs = jnp.cumsum(vec)
                excl = (cs - vec + tot).astype(jnp.int32)
                flag = jnp.where(vec >= 2.0, one_i, zero_i)
                bcum[pl.ds(i * L, L)] = excl * 2 + flag
                return tot + jnp.sum(vec)
            lax.fori_loop(0, K // L, prefix, jnp.float32(0))

            pltpu.sync_copy(bcum, lut_shared.at[sid])
        plsc.subcore_barrier()

        def batch_body(b, _):
            pltpu.sync_copy(lut_shared.at[b], bcum)
            pltpu.sync_copy(bins_hbm.at[b], binspad.at[pl.ds(0, T)])
            binspad[pl.ds(T, L)] = jnp.full((L,), 2.0, jnp.float32)

            def zero_hist(i, _):
                hist[pl.ds(i * L, L)] = zf16
                return 0
            lax.fori_loop(0, HIST_LEN // L, zero_hist, 0)

            wrow = wid * (PER_W // N)

            def start(s):
                slot = s % 2
                r0 = wrow + s * CROWS
                hv = pltpu.async_copy(
                    inp4_hbm.at[b, pl.ds(r0, CROWS)], vbufs[slot], svs[slot])
                hw = pltpu.async_copy(
                    w4_hbm.at[pl.ds(b * N + r0, CROWS)], wbufs[slot],
                    sws[slot])
                return hv, hw

            minvs = [jnp.full((L,), 2.0, jnp.float32) for _ in range(U)]
            idxvs = [jnp.zeros((L,), jnp.float32) for _ in range(U)]
            pend = start(0)
            for s in range(NCHUNK):
                slot = s % 2
                nxt = start(s + 1) if s + 1 < NCHUNK else None
                pend[0].wait()
                pend[1].wait()
                pend = nxt
                vb = vbufs[slot]
                wb = wbufs[slot]
                cbase = ((wrow + s * CROWS) * N).astype(jnp.float32)

                def body(kk, carry):
                    mvs = list(carry[:U])
                    ivs = list(carry[U:])
                    r = kk // (N // (L * U))
                    c0 = (kk % (N // (L * U))) * (L * U)
                    rvec = jnp.full((L,), 0, jnp.int32) + r
                    vs, ts = [], []
                    flags = None
                    for u in range(U):
                        v = vb[r, pl.ds(c0 + u * L, L)]
                        c = jnp.minimum((v * K).astype(jnp.int32), K - 1)
                        raw = plsc.load_gather(bcum, [c])
                        t = lax.shift_right_logical(raw, 1)
                        fl = lax.bitwise_and(raw, one_i)
                        g = plsc.load_gather(binspad, [t])
                        t = t + jnp.where(g <= v, one_i, zero_i)
                        vs.append(v)
                        ts.append(t)
                        flags = fl if flags is None else flags | fl
                    need_more = jnp.any(flags > 0)

                    def slow(tts):
                        def wcond(tts):
                            m = None
                            for u in range(U):
                                mu = (plsc.load_gather(binspad, [tts[u]])
                                      <= vs[u])
                                m = mu if m is None else m | mu
                            return jnp.any(m)

                        def wbody(tts):
                            return tuple(
                                tts[u] + jnp.where(
                                    plsc.load_gather(binspad, [tts[u]])
                                    <= vs[u], one_i, zero_i)
                                for u in range(U))
                        return lax.while_loop(wcond, wbody, tts)

                    ts = list(lax.cond(
                        need_more, slow, lambda tts: tts, tuple(ts)))

                    for u in range(U):
                        cvec = c0 + u * L + lanes
                        li = r * N + cvec
                        cw = cvec * F
                        wf0 = plsc.load_gather(wb, [rvec, cw])
                        wf1 = plsc.load_gather(wb, [rvec, cw + 1])
                        t2 = ts[u] * 2
                        plsc.addupdate_scatter(hist, [t2], wf0)
                        plsc.addupdate_scatter(hist, [t2 + 1], wf1)
                        m = vs[u] < mvs[u]
                        gidx = cbase + li.astype(jnp.float32)
                        mvs[u] = jnp.where(m, vs[u], mvs[u])
                        ivs[u] = jnp.where(m, gidx, ivs[u])
                    return tuple(mvs) + tuple(ivs)

                carry = lax.fori_loop(
                    0, CH // (L * U), body, tuple(minvs) + tuple(idxvs),
                    unroll=2)
                minvs = list(carry[:U])
                idxvs = list(carry[U:])

            minv, idxv = minvs[0], idxvs[0]
            for u in range(1, U):
                better = (minvs[u] < minv) | (
                    (minvs[u] == minv) & (idxvs[u] < idxv))
                minv = jnp.where(better, minvs[u], minv)
                idxv = jnp.where(better, idxvs[u], idxv)

            minbuf[pl.ds(0, L)] = minv
            minbuf[pl.ds(L, L)] = idxv
            pltpu.sync_copy(hist, out_hist.at[wid, b])
            pltpu.sync_copy(minbuf, out_min.at[wid, b])
            return 0

        lax.fori_loop(0, B, batch_body, 0)

    return k(inp_flat, bins, w_flat)


def kernel(input, bins, weights):
    hist_parts, min_parts = _sc_histogram(
        input, bins, weights.reshape(B * N, N * F))

    hist = hist_parts.reshape(NW, B, HIST_LEN // F, F).sum(axis=0)
    hist = hist[:, : T + 1, :]

    vals = min_parts[:, :, :L]
    idxs = min_parts[:, :, L:]
    minval = vals.min(axis=(0, 2))
    cand = jnp.where(vals == minval[None, :, None], idxs, jnp.inf)
    minidx = cand.min(axis=(0, 2)).astype(jnp.int32)
    barange = jnp.arange(B)
    wmin = weights[barange, minidx // 1024, minidx % 1024, :]
    bkt_min = jax.vmap(
        lambda bb, mv: jnp.searchsorted(bb, mv, side="right"))(bins, minval)

    hist = hist.at[barange, bkt_min, :].add(-wmin)
    hist = hist.at[:, 0, :].add(wmin)
    return hist[:, :T, :]


# speedup vs baseline: 1.0824x; 1.0824x over previous
import functools

import jax
import jax.numpy as jnp
from jax import lax
from jax.experimental import pallas as pl
from jax.experimental.pallas import tpu as pltpu
from jax.experimental.pallas import tpu_sc as plsc

NC = 2
NS = 16
NW = NC * NS
L = 16

B = 8
M = 1024 * 1024
T = 512
F = 2

K = 8192
N = 1024
PER_W = M // NW
CH = 8192
CROWS = CH // N
NCHUNK = PER_W // CH
U = 4
BINS_PAD = T + L
HIST_LEN = 1056
NB_LUT = 8


def _sc_histogram(inp_flat, bins, w_flat):
    mesh = plsc.VectorSubcoreMesh(
        core_axis_name="c", subcore_axis_name="s", num_cores=NC,
        num_subcores=NS)

    @functools.partial(
        pl.kernel,
        out_type=[
            jax.ShapeDtypeStruct((NW, B, HIST_LEN), jnp.float32),
            jax.ShapeDtypeStruct((NW, B, 2 * L), jnp.float32),
        ],
        mesh=mesh,
        compiler_params=pltpu.CompilerParams(needs_layout_passes=False),
        scratch_types=[
            pltpu.VMEM((CROWS, N), jnp.float32),
            pltpu.VMEM((CROWS, N), jnp.float32),
            pltpu.VMEM((CROWS, N * F), jnp.float32),
            pltpu.VMEM((CROWS, N * F), jnp.float32),
            pltpu.VMEM((BINS_PAD,), jnp.float32),
            pltpu.VMEM((K,), jnp.float32),
            pltpu.VMEM((K,), jnp.int32),
            pltpu.VMEM((HIST_LEN,), jnp.float32),
            pltpu.VMEM((2 * L,), jnp.float32),
            pltpu.VMEM_SHARED((B, K), jnp.int32),
            pltpu.SemaphoreType.DMA,
            pltpu.SemaphoreType.DMA,
            pltpu.SemaphoreType.DMA,
            pltpu.SemaphoreType.DMA,
        ],
    )
    def k(inp4_hbm, bins_hbm, w4_hbm, out_hist, out_min,
          vbuf0, vbuf1, wbuf0, wbuf1, binspad, bcnt, bcum, hist, minbuf,
          lut_shared, sv0, sv1, sw0, sw1):
        cid = lax.axis_index("c")
        sid = lax.axis_index("s")
        wid = sid * NC + cid
        wbase = wid * PER_W
        lanes = lax.iota(jnp.int32, L)
        zf16 = jnp.zeros((L,), jnp.float32)
        ones_f = jnp.ones((L,), jnp.float32)
        one_i = jnp.ones((L,), jnp.int32)
        zero_i = jnp.zeros((L,), jnp.int32)
        vbufs = (vbuf0, vbuf1)
        wbufs = (wbuf0, wbuf1)
        svs = (sv0, sv1)
        sws = (sw0, sw1)

        @pl.when(sid < NB_LUT)
        def _build():
            def zero_lut(i, _):
                bcnt[pl.ds(i * L, L)] = zf16
                return 0
            lax.fori_loop(0, K // L, zero_lut, 0)

            pltpu.sync_copy(bins_hbm.at[sid], binspad.at[pl.ds(0, T)])
            binspad[pl.ds(T, L)] = jnp.full((L,), 2.0, jnp.float32)

            def bin_cells(i, _):
                bv = binspad[pl.ds(i * L, L)]
                cb = jnp.minimum((bv * K).astype(jnp.int32), K - 1)
                plsc.addupdate_scatter(bcnt, [cb], ones_f)
                return 0
            lax.fori_loop(0, T // L, bin_cells, 0)

            def prefix(i, tot):
                vec = bcnt[pl.ds(i * L, L)]
                cs = jnp.cumsum(vec)
                excl = (cs - vec + tot).astype(jnp.int32)
                flag = jnp.where(vec >= 2.0, one_i, zero_i)
                bcum[pl.ds(i * L, L)] = excl * 2 + flag
                return tot + jnp.sum(vec)
            lax.fori_loop(0, K // L, prefix, jnp.float32(0))

            pltpu.sync_copy(bcum, lut_shared.at[sid])
        plsc.subcore_barrier()

        def batch_body(b, _):
            pltpu.sync_copy(lut_shared.at[b], bcum)
            pltpu.sync_copy(bins_hbm.at[b], binspad.at[pl.ds(0, T)])
            binspad[pl.ds(T, L)] = jnp.full((L,), 2.0, jnp.float32)

            def zero_hist(i, _):
                hist[pl.ds(i * L, L)] = zf16
                return 0
            lax.fori_loop(0, HIST_LEN // L, zero_hist, 0)

            wrow = wid * (PER_W // N)

            def start(s):
                slot = s % 2
                r0 = wrow + s * CROWS
                hv = pltpu.async_copy(
                    inp4_hbm.at[b, pl.ds(r0, CROWS)], vbufs[slot], svs[slot])
                hw = pltpu.async_copy(
                    w4_hbm.at[pl.ds(b * N + r0, CROWS)], wbufs[slot],
                    sws[slot])
                return hv, hw

            minvs = [jnp.full((L,), 2.0, jnp.float32) for _ in range(U)]
            idxvs = [jnp.zeros((L,), jnp.float32) for _ in range(U)]
            pend = start(0)
            for s in range(NCHUNK):
                slot = s % 2
                nxt = start(s + 1) if s + 1 < NCHUNK else None
                pend[0].wait()
                pend[1].wait()
                pend = nxt
                vb = vbufs[slot]
                wb = wbufs[slot]
                cbase = ((wrow + s * CROWS) * N).astype(jnp.float32)

                def body(kk, carry):
                    mvs = list(carry[:U])
                    ivs = list(carry[U:])
                    r = kk // (N // (L * U))
                    c0 = (kk % (N // (L * U))) * (L * U)
                    rvec = jnp.full((L,), 0, jnp.int32) + r
                    vs, ts = [], []
                    flags = None
                    for u in range(U):
                        v = vb[r, pl.ds(c0 + u * L, L)]
                        c = jnp.minimum((v * K).astype(jnp.int32), K - 1)
                        raw = plsc.load_gather(bcum, [c])
                        t = lax.shift_right_logical(raw, 1)
                        fl = lax.bitwise_and(raw, one_i)
                        g = plsc.load_gather(binspad, [t])
                        t = t + jnp.where(g <= v, one_i, zero_i)
                        vs.append(v)
                        ts.append(t)
                        flags = fl if flags is None else flags | fl
                    need_more = jnp.any(flags > 0)

                    def slow(tts):
                        def wcond(tts):
                            m = None
                            for u in range(U):
                                mu = (plsc.load_gather(binspad, [tts[u]])
                                      <= vs[u])
                                m = mu if m is None else m | mu
                            return jnp.any(m)

                        def wbody(tts):
                            return tuple(
                                tts[u] + jnp.where(
                                    plsc.load_gather(binspad, [tts[u]])
                                    <= vs[u], one_i, zero_i)
                                for u in range(U))
                        return lax.while_loop(wcond, wbody, tts)

                    ts = list(lax.cond(
                        need_more, slow, lambda tts: tts, tuple(ts)))

                    for u in range(U):
                        cvec = c0 + u * L + lanes
                        li = r * N + cvec
                        cw = cvec * F
                        wf0 = plsc.load_gather(wb, [rvec, cw])
                        wf1 = plsc.load_gather(wb, [rvec, cw + 1])
                        t2 = ts[u] * 2
                        plsc.addupdate_scatter(hist, [t2], wf0)
                        plsc.addupdate_scatter(hist, [t2 + 1], wf1)
                        m = vs[u] < mvs[u]
                        gidx = cbase + li.astype(jnp.float32)
                        mvs[u] = jnp.where(m, vs[u], mvs[u])
                        ivs[u] = jnp.where(m, gidx, ivs[u])
                    return tuple(mvs) + tuple(ivs)

                carry = plsc.parallel_loop(
                    0, CH // (L * U), 1, unroll=2,
                    carry=tuple(minvs) + tuple(idxvs))(body)
                minvs = list(carry[:U])
                idxvs = list(carry[U:])

            minv, idxv = minvs[0], idxvs[0]
            for u in range(1, U):
                better = (minvs[u] < minv) | (
                    (minvs[u] == minv) & (idxvs[u] < idxv))
                minv = jnp.where(better, minvs[u], minv)
                idxv = jnp.where(better, idxvs[u], idxv)

            minbuf[pl.ds(0, L)] = minv
            minbuf[pl.ds(L, L)] = idxv
            pltpu.sync_copy(hist, out_hist.at[wid, b])
            pltpu.sync_copy(minbuf, out_min.at[wid, b])
            return 0

        lax.fori_loop(0, B, batch_body, 0)

    return k(inp_flat, bins, w_flat)


def kernel(input, bins, weights):
    hist_parts, min_parts = _sc_histogram(
        input, bins, weights.reshape(B * N, N * F))

    hist = hist_parts.reshape(NW, B, HIST_LEN // F, F).sum(axis=0)
    hist = hist[:, : T + 1, :]

    vals = min_parts[:, :, :L]
    idxs = min_parts[:, :, L:]
    minval = vals.min(axis=(0, 2))
    cand = jnp.where(vals == minval[None, :, None], idxs, jnp.inf)
    minidx = cand.min(axis=(0, 2)).astype(jnp.int32)
    barange = jnp.arange(B)
    wmin = weights[barange, minidx // 1024, minidx % 1024, :]
    bkt_min = jax.vmap(
        lambda bb, mv: jnp.searchsorted(bb, mv, side="right"))(bins, minval)

    hist = hist.at[barange, bkt_min, :].add(-wmin)
    hist = hist.at[:, 0, :].add(wmin)
    return hist[:, :T, :]


# speedup vs baseline: 1.1043x; 1.0203x over previous
import functools

import jax
import jax.numpy as jnp
from jax import lax
from jax.experimental import pallas as pl
from jax.experimental.pallas import tpu as pltpu
from jax.experimental.pallas import tpu_sc as plsc

NC = 2
NS = 16
NW = NC * NS
L = 16

B = 8
M = 1024 * 1024
T = 512
F = 2

K = 8192
N = 1024
PER_W = M // NW
CH = 8192
CROWS = CH // N
NCHUNK = PER_W // CH
U = 4
BINS_PAD = T + L
HIST_LEN = 1056
NB_LUT = 8


def _sc_histogram(inp_flat, bins, w_flat):
    mesh = plsc.VectorSubcoreMesh(
        core_axis_name="c", subcore_axis_name="s", num_cores=NC,
        num_subcores=NS)

    @functools.partial(
        pl.kernel,
        out_type=[
            jax.ShapeDtypeStruct((NW, B, HIST_LEN), jnp.float32),
            jax.ShapeDtypeStruct((NW, B, 2 * L), jnp.float32),
        ],
        mesh=mesh,
        compiler_params=pltpu.CompilerParams(needs_layout_passes=False),
        scratch_types=[
            pltpu.VMEM((CROWS, N), jnp.float32),
            pltpu.VMEM((CROWS, N), jnp.float32),
            pltpu.VMEM((CROWS, N * F), jnp.float32),
            pltpu.VMEM((CROWS, N * F), jnp.float32),
            pltpu.VMEM((BINS_PAD,), jnp.float32),
            pltpu.VMEM((K,), jnp.float32),
            pltpu.VMEM((K,), jnp.int32),
            pltpu.VMEM((HIST_LEN,), jnp.float32),
            pltpu.VMEM((2 * L,), jnp.float32),
            pltpu.VMEM_SHARED((B, K), jnp.int32),
            pltpu.SemaphoreType.DMA,
            pltpu.SemaphoreType.DMA,
            pltpu.SemaphoreType.DMA,
            pltpu.SemaphoreType.DMA,
        ],
    )
    def k(inp4_hbm, bins_hbm, w4_hbm, out_hist, out_min,
          vbuf0, vbuf1, wbuf0, wbuf1, binspad, bcnt, bcum, hist, minbuf,
          lut_shared, sv0, sv1, sw0, sw1):
        cid = lax.axis_index("c")
        sid = lax.axis_index("s")
        wid = sid * NC + cid
        wbase = wid * PER_W
        lanes = lax.iota(jnp.int32, L)
        zf16 = jnp.zeros((L,), jnp.float32)
        ones_f = jnp.ones((L,), jnp.float32)
        one_i = jnp.ones((L,), jnp.int32)
        zero_i = jnp.zeros((L,), jnp.int32)
        vbufs = (vbuf0, vbuf1)
        wbufs = (wbuf0, wbuf1)
        svs = (sv0, sv1)
        sws = (sw0, sw1)

        @pl.when(sid < NB_LUT)
        def _build():
            def zero_lut(i, _):
                bcnt[pl.ds(i * L, L)] = zf16
                return 0
            lax.fori_loop(0, K // L, zero_lut, 0)

            pltpu.sync_copy(bins_hbm.at[sid], binspad.at[pl.ds(0, T)])
            binspad[pl.ds(T, L)] = jnp.full((L,), 2.0, jnp.float32)

            def bin_cells(i, _):
                bv = binspad[pl.ds(i * L, L)]
                cb = jnp.minimum((bv * K).astype(jnp.int32), K - 1)
                plsc.addupdate_scatter(bcnt, [cb], ones_f)
                return 0
            lax.fori_loop(0, T // L, bin_cells, 0)

            def prefix(i, tot):
                vec = bcnt[pl.ds(i * L, L)]
                cs = jnp.cumsum(vec)
                excl = (cs - vec + tot).astype(jnp.int32)
                flag = jnp.where(vec >= 2.0, one_i, zero_i)
                bcum[pl.ds(i * L, L)] = excl * 2 + flag
                return tot + jnp.sum(vec)
            lax.fori_loop(0, K // L, prefix, jnp.float32(0))

            pltpu.sync_copy(bcum, lut_shared.at[sid])
        plsc.subcore_barrier()

        def batch_body(b, _):
            pltpu.sync_copy(lut_shared.at[b], bcum)
            pltpu.sync_copy(bins_hbm.at[b], binspad.at[pl.ds(0, T)])
            binspad[pl.ds(T, L)] = jnp.full((L,), 2.0, jnp.float32)

            def zero_hist(i, _):
                hist[pl.ds(i * L, L)] = zf16
                return 0
            lax.fori_loop(0, HIST_LEN // L, zero_hist, 0)

            wrow = wid * (PER_W // N)

            def start(s):
                slot = s % 2
                r0 = wrow + s * CROWS
                hv = pltpu.async_copy(
                    inp4_hbm.at[b, pl.ds(r0, CROWS)], vbufs[slot], svs[slot])
                hw = pltpu.async_copy(
                    w4_hbm.at[pl.ds(b * N + r0, CROWS)], wbufs[slot],
                    sws[slot])
                return hv, hw

            minvs = [jnp.full((L,), 2.0, jnp.float32) for _ in range(U)]
            idxvs = [jnp.zeros((L,), jnp.float32) for _ in range(U)]
            pend = start(0)
            for s in range(NCHUNK):
                slot = s % 2
                nxt = start(s + 1) if s + 1 < NCHUNK else None
                pend[0].wait()
                pend[1].wait()
                pend = nxt
                vb = vbufs[slot]
                wb = wbufs[slot]
                cbase = ((wrow + s * CROWS) * N).astype(jnp.float32)

                def body(kk, carry):
                    mvs = list(carry[:U])
                    ivs = list(carry[U:])
                    r = kk // (N // (L * U))
                    c0 = (kk % (N // (L * U))) * (L * U)
                    rvec = jnp.full((L,), 0, jnp.int32) + r
                    vs, ts = [], []
                    flags = None
                    for u in range(U):
                        v = vb[r, pl.ds(c0 + u * L, L)]
                        c = jnp.minimum((v * K).astype(jnp.int32), K - 1)
                        raw = plsc.load_gather(bcum, [c])
                        t = lax.shift_right_logical(raw, 1)
                        fl = lax.bitwise_and(raw, one_i)
                        g = plsc.load_gather(binspad, [t])
                        t = t + jnp.where(g <= v, one_i, zero_i)
                        vs.append(v)
                        ts.append(t)
                        flags = fl if flags is None else flags | fl
                    need_more = jnp.any(flags > 0)

                    def slow(tts):
                        def wcond(tts):
                            m = None
                            for u in range(U):
                                mu = (plsc.load_gather(binspad, [tts[u]])
                                      <= vs[u])
                                m = mu if m is None else m | mu
                            return jnp.any(m)

                        def wbody(tts):
                            return tuple(
                                tts[u] + jnp.where(
                                    plsc.load_gather(binspad, [tts[u]])
                                    <= vs[u], one_i, zero_i)
                                for u in range(U))
                        return lax.while_loop(wcond, wbody, tts)

                    ts = list(lax.cond(
                        need_more, slow, lambda tts: tts, tuple(ts)))

                    for u in range(U):
                        cvec = c0 + u * L + lanes
                        li = r * N + cvec
                        cw = cvec * F
                        wf0 = plsc.load_gather(wb, [rvec, cw])
                        wf1 = plsc.load_gather(wb, [rvec, cw + 1])
                        t2 = ts[u] * 2
                        plsc.addupdate_scatter(hist, [t2], wf0)
                        plsc.addupdate_scatter(hist, [t2 + 1], wf1)
                        m = vs[u] < mvs[u]
                        gidx = cbase + li.astype(jnp.float32)
                        mvs[u] = jnp.where(m, vs[u], mvs[u])
                        ivs[u] = jnp.where(m, gidx, ivs[u])
                    return tuple(mvs) + tuple(ivs)

                carry = plsc.parallel_loop(
                    0, CH // (L * U), 1, unroll=4,
                    carry=tuple(minvs) + tuple(idxvs))(body)
                minvs = list(carry[:U])
                idxvs = list(carry[U:])

            minv, idxv = minvs[0], idxvs[0]
            for u in range(1, U):
                better = (minvs[u] < minv) | (
                    (minvs[u] == minv) & (idxvs[u] < idxv))
                minv = jnp.where(better, minvs[u], minv)
                idxv = jnp.where(better, idxvs[u], idxv)

            minbuf[pl.ds(0, L)] = minv
            minbuf[pl.ds(L, L)] = idxv
            pltpu.sync_copy(hist, out_hist.at[wid, b])
            pltpu.sync_copy(minbuf, out_min.at[wid, b])
            return 0

        lax.fori_loop(0, B, batch_body, 0)

    return k(inp_flat, bins, w_flat)


def kernel(input, bins, weights):
    hist_parts, min_parts = _sc_histogram(
        input, bins, weights.reshape(B * N, N * F))

    hist = hist_parts.reshape(NW, B, HIST_LEN // F, F).sum(axis=0)
    hist = hist[:, : T + 1, :]

    vals = min_parts[:, :, :L]
    idxs = min_parts[:, :, L:]
    minval = vals.min(axis=(0, 2))
    cand = jnp.where(vals == minval[None, :, None], idxs, jnp.inf)
    minidx = cand.min(axis=(0, 2)).astype(jnp.int32)
    barange = jnp.arange(B)
    wmin = weights[barange, minidx // 1024, minidx % 1024, :]
    bkt_min = jax.vmap(
        lambda bb, mv: jnp.searchsorted(bb, mv, side="right"))(bins, minval)

    hist = hist.at[barange, bkt_min, :].add(-wmin)
    hist = hist.at[:, 0, :].add(wmin)
    return hist[:, :T, :]
